# Initial kernel scaffold; baseline (speedup 1.0000x reference)
import jax
import jax.numpy as jnp
from jax.experimental import pallas as pl


def kernel(x, edge_index, batch, Wl1, bl1, Wr1, br1, Wl2, bl2, Wr2, br2):
    raise NotImplementedError("write your pallas kernel here")


# speedup vs baseline: 3.3536x; 3.3536x over previous
import jax
import jax.numpy as jnp
from jax import lax
from jax.experimental import pallas as pl
from jax.experimental.pallas import tpu as pltpu
from jax.experimental.pallas import tpu_sc as plsc

D = 128
NS = 16
NC = 2
NW = NC * NS
CB = 128
L = 16


def _acc_rows(n_nodes):
    return ((n_nodes + NS * CB) // (NS * CB)) * (NS * CB)


def _fill(ref, value):
    def st(t, carry):
        ref[t // (D // L), pl.ds((t % (D // L)) * L, L)] = (
            jnp.full((L,), value, jnp.float32))
        return carry

    lax.fori_loop(0, CB * (D // L), st, 0)


def _make_agg(n_nodes, npw):
    acc_rows = _acc_rows(n_nodes)
    zrows = acc_rows // NS
    nslab = zrows // CB

    mesh = plsc.VectorSubcoreMesh(core_axis_name="c", subcore_axis_name="s")

    def body(x_hbm, src_hbm, dst_hbm, out_hbm, idx_sb, idx_db, rows, acc,
             sem):
        c = lax.axis_index("c")
        s = lax.axis_index("s")
        w = c * NS + s

        _fill(rows, 0.0)
        z0 = s * zrows
        for k in range(nslab):
            pltpu.sync_copy(rows, acc.at[pl.ds(z0 + k * CB, CB)])
        plsc.subcore_barrier()

        for t in range(npw):
            base = (w * npw + t) * CB
            pltpu.sync_copy(src_hbm.at[pl.ds(base, CB)], idx_sb)
            pltpu.sync_copy(dst_hbm.at[pl.ds(base, CB)], idx_db)
            pltpu.async_copy(x_hbm.at[idx_sb], rows, sem).wait()
            pltpu.sync_copy(rows, acc.at[idx_db], add=True)
        plsc.subcore_barrier()
        r0 = c * acc_rows + s * zrows
        for k in range(nslab):
            pltpu.sync_copy(acc.at[pl.ds(z0 + k * CB, CB)], rows)
            pltpu.sync_copy(rows, out_hbm.at[pl.ds(r0 + k * CB, CB)])

    return pl.kernel(
        body,
        out_type=jax.ShapeDtypeStruct((NC * acc_rows, D), jnp.float32),
        mesh=mesh,
        scratch_types=[
            pltpu.VMEM((CB,), jnp.int32),
            pltpu.VMEM((CB,), jnp.int32),
            pltpu.VMEM((CB, D), jnp.float32),
            pltpu.VMEM_SHARED((acc_rows, D), jnp.float32),
            pltpu.SemaphoreType.DMA,
        ]), acc_rows


def _make_cnt(n_nodes, npw):
    acc_rows = _acc_rows(n_nodes)
    zrows = acc_rows // NS
    nslab = zrows // CB

    mesh = plsc.VectorSubcoreMesh(core_axis_name="c", subcore_axis_name="s")

    def body(dst_hbm, out_hbm, idx_db, rows, acc, sem):
        c = lax.axis_index("c")
        s = lax.axis_index("s")
        w = c * NS + s

        _fill(rows, 0.0)
        z0 = s * zrows
        for k in range(nslab):
            pltpu.sync_copy(rows, acc.at[pl.ds(z0 + k * CB, CB)])
        _fill(rows, 1.0)
        plsc.subcore_barrier()

        for t in range(npw):
            base = (w * npw + t) * CB
            pltpu.sync_copy(dst_hbm.at[pl.ds(base, CB)], idx_db)
            pltpu.sync_copy(rows, acc.at[idx_db], add=True)
        plsc.subcore_barrier()
        r0 = c * acc_rows + s * zrows
        for k in range(nslab):
            pltpu.sync_copy(acc.at[pl.ds(z0 + k * CB, CB)], rows)
            pltpu.sync_copy(rows, out_hbm.at[pl.ds(r0 + k * CB, CB)])
        del sem

    return pl.kernel(
        body,
        out_type=jax.ShapeDtypeStruct((NC * acc_rows, D), jnp.float32),
        mesh=mesh,
        scratch_types=[
            pltpu.VMEM((CB,), jnp.int32),
            pltpu.VMEM((CB, D), jnp.float32),
            pltpu.VMEM_SHARED((acc_rows, D), jnp.float32),
            pltpu.SemaphoreType.DMA,
        ])


def _make_combine(n_nodes, rblk):
    grid = (n_nodes // rblk,)

    def body(p0_ref, p1_ref, c0_ref, c1_ref, x_ref, wl_ref, wr_ref, b_ref,
             o_ref):
        cnt = c0_ref[:, 0:1] + c1_ref[:, 0:1]
        mean = (p0_ref[...] + p1_ref[...]) / jnp.maximum(cnt, 1.0)
        h = (jnp.dot(mean, wl_ref[...], preferred_element_type=jnp.float32)
             + jnp.dot(x_ref[...], wr_ref[...],
                       preferred_element_type=jnp.float32)
             + b_ref[...])
        o_ref[...] = jnp.where(h >= 0.0, h, 0.01 * h) + x_ref[...]

    return pl.pallas_call(
        body,
        grid=grid,
        in_specs=[
            pl.BlockSpec((rblk, D), lambda i: (i, 0)),
            pl.BlockSpec((rblk, D), lambda i: (i, 0)),
            pl.BlockSpec((rblk, D), lambda i: (i, 0)),
            pl.BlockSpec((rblk, D), lambda i: (i, 0)),
            pl.BlockSpec((rblk, D), lambda i: (i, 0)),
            pl.BlockSpec((D, D), lambda i: (0, 0)),
            pl.BlockSpec((D, D), lambda i: (0, 0)),
            pl.BlockSpec((1, D), lambda i: (0, 0)),
        ],
        out_specs=pl.BlockSpec((rblk, D), lambda i: (i, 0)),
        out_shape=jax.ShapeDtypeStruct((n_nodes, D), jnp.float32),
    )


def kernel(x, edge_index, batch, Wl1, bl1, Wr1, br1, Wl2, bl2, Wr2, br2):
    n, d = x.shape
    assert d == D and n % NS == 0
    e = edge_index.shape[1]

    epw = -(-e // (NW * CB)) * CB
    ep = epw * NW
    npw = epw // CB
    pad = ep - e
    src_p = jnp.concatenate([edge_index[0], jnp.zeros((pad,), jnp.int32)])
    dst_p = jnp.concatenate([edge_index[1], jnp.full((pad,), n, jnp.int32)])

    agg, acc_rows = _make_agg(n, npw)
    cntk = _make_cnt(n, npw)
    comb = _make_combine(n, 1000)

    b1 = (bl1 + br1).reshape(1, D)
    b2 = (bl2 + br2).reshape(1, D)

    cnt = cntk(dst_p).reshape(NC, acc_rows, D)
    p1 = agg(x, src_p, dst_p).reshape(NC, acc_rows, D)
    y1 = comb(p1[0, :n], p1[1, :n], cnt[0, :n], cnt[1, :n], x,
              Wl1.T, Wr1.T, b1)
    p2 = agg(y1, src_p, dst_p).reshape(NC, acc_rows, D)
    y2 = comb(p2[0, :n], p2[1, :n], cnt[0, :n], cnt[1, :n], y1,
              Wl2.T, Wr2.T, b2)
    return (y2, edge_index, batch)


# speedup vs baseline: 4.1868x; 1.2484x over previous
import jax
import jax.numpy as jnp
from jax import lax
from jax.experimental import pallas as pl
from jax.experimental.pallas import tpu as pltpu
from jax.experimental.pallas import tpu_sc as plsc

D = 128
NS = 16
NC = 2
NW = NC * NS
CB = 128
L = 16


def _acc_rows(n_nodes):
    return ((n_nodes + NS * CB) // (NS * CB)) * (NS * CB)


def _fill(ref, value):
    def st(t, carry):
        ref[t // (D // L), pl.ds((t % (D // L)) * L, L)] = (
            jnp.full((L,), value, jnp.float32))
        return carry

    lax.fori_loop(0, CB * (D // L), st, 0)


def _make_agg(n_nodes, npw):
    acc_rows = _acc_rows(n_nodes)
    zrows = acc_rows // NS
    nslab = zrows // CB

    mesh = plsc.VectorSubcoreMesh(core_axis_name="c", subcore_axis_name="s")

    def body(x_hbm, src_hbm, dst_hbm, out_hbm, sb0, sb1, db0, db1, rows0,
             rows1, acc, sem0, sem1):
        c = lax.axis_index("c")
        s = lax.axis_index("s")
        w = c * NS + s
        sb = (sb0, sb1)
        db = (db0, db1)
        rows = (rows0, rows1)
        sem = (sem0, sem1)

        _fill(rows0, 0.0)
        z0 = s * zrows
        for k in range(nslab):
            pltpu.sync_copy(rows0, acc.at[pl.ds(z0 + k * CB, CB)])
        plsc.subcore_barrier()

        base = w * npw * CB
        pltpu.sync_copy(src_hbm.at[pl.ds(base, CB)], sb0)
        pltpu.sync_copy(dst_hbm.at[pl.ds(base, CB)], db0)
        pend = pltpu.async_copy(x_hbm.at[sb0], rows0, sem0)
        for t in range(npw):
            cur = t % 2
            nxt = (t + 1) % 2
            if t + 1 < npw:
                base = (w * npw + t + 1) * CB
                pltpu.sync_copy(src_hbm.at[pl.ds(base, CB)], sb[nxt])
                pltpu.sync_copy(dst_hbm.at[pl.ds(base, CB)], db[nxt])
                nxt_pend = pltpu.async_copy(x_hbm.at[sb[nxt]], rows[nxt],
                                            sem[nxt])
            pend.wait()
            pltpu.sync_copy(rows[cur], acc.at[db[cur]], add=True)
            if t + 1 < npw:
                pend = nxt_pend
        plsc.subcore_barrier()
        r0 = c * acc_rows + s * zrows
        for k in range(nslab):
            pltpu.sync_copy(acc.at[pl.ds(z0 + k * CB, CB)], rows0)
            pltpu.sync_copy(rows0, out_hbm.at[pl.ds(r0 + k * CB, CB)])

    return pl.kernel(
        body,
        out_type=jax.ShapeDtypeStruct((NC * acc_rows, D), jnp.float32),
        mesh=mesh,
        scratch_types=[
            pltpu.VMEM((CB,), jnp.int32),
            pltpu.VMEM((CB,), jnp.int32),
            pltpu.VMEM((CB,), jnp.int32),
            pltpu.VMEM((CB,), jnp.int32),
            pltpu.VMEM((CB, D), jnp.float32),
            pltpu.VMEM((CB, D), jnp.float32),
            pltpu.VMEM_SHARED((acc_rows, D), jnp.float32),
            pltpu.SemaphoreType.DMA,
            pltpu.SemaphoreType.DMA,
        ]), acc_rows


def _make_cnt(n_nodes, npw):
    acc_rows = _acc_rows(n_nodes)
    zrows = acc_rows // NS
    nslab = zrows // CB

    mesh = plsc.VectorSubcoreMesh(core_axis_name="c", subcore_axis_name="s")

    def body(dst_hbm, out_hbm, idx_db, rows, acc, sem):
        c = lax.axis_index("c")
        s = lax.axis_index("s")
        w = c * NS + s

        _fill(rows, 0.0)
        z0 = s * zrows
        for k in range(nslab):
            pltpu.sync_copy(rows, acc.at[pl.ds(z0 + k * CB, CB)])
        _fill(rows, 1.0)
        plsc.subcore_barrier()

        for t in range(npw):
            base = (w * npw + t) * CB
            pltpu.sync_copy(dst_hbm.at[pl.ds(base, CB)], idx_db)
            pltpu.sync_copy(rows, acc.at[idx_db], add=True)
        plsc.subcore_barrier()
        r0 = c * acc_rows + s * zrows
        for k in range(nslab):
            pltpu.sync_copy(acc.at[pl.ds(z0 + k * CB, CB)], rows)
            pltpu.sync_copy(rows, out_hbm.at[pl.ds(r0 + k * CB, CB)])
        del sem

    return pl.kernel(
        body,
        out_type=jax.ShapeDtypeStruct((NC * acc_rows, D), jnp.float32),
        mesh=mesh,
        scratch_types=[
            pltpu.VMEM((CB,), jnp.int32),
            pltpu.VMEM((CB, D), jnp.float32),
            pltpu.VMEM_SHARED((acc_rows, D), jnp.float32),
            pltpu.SemaphoreType.DMA,
        ])


def _make_combine(n_nodes, rblk):
    grid = (n_nodes // rblk,)

    def body(p0_ref, p1_ref, c0_ref, c1_ref, x_ref, wl_ref, wr_ref, b_ref,
             o_ref):
        cnt = c0_ref[:, 0:1] + c1_ref[:, 0:1]
        mean = (p0_ref[...] + p1_ref[...]) / jnp.maximum(cnt, 1.0)
        h = (jnp.dot(mean, wl_ref[...], preferred_element_type=jnp.float32)
             + jnp.dot(x_ref[...], wr_ref[...],
                       preferred_element_type=jnp.float32)
             + b_ref[...])
        o_ref[...] = jnp.where(h >= 0.0, h, 0.01 * h) + x_ref[...]

    return pl.pallas_call(
        body,
        grid=grid,
        in_specs=[
            pl.BlockSpec((rblk, D), lambda i: (i, 0)),
            pl.BlockSpec((rblk, D), lambda i: (i, 0)),
            pl.BlockSpec((rblk, D), lambda i: (i, 0)),
            pl.BlockSpec((rblk, D), lambda i: (i, 0)),
            pl.BlockSpec((rblk, D), lambda i: (i, 0)),
            pl.BlockSpec((D, D), lambda i: (0, 0)),
            pl.BlockSpec((D, D), lambda i: (0, 0)),
            pl.BlockSpec((1, D), lambda i: (0, 0)),
        ],
        out_specs=pl.BlockSpec((rblk, D), lambda i: (i, 0)),
        out_shape=jax.ShapeDtypeStruct((n_nodes, D), jnp.float32),
    )


def kernel(x, edge_index, batch, Wl1, bl1, Wr1, br1, Wl2, bl2, Wr2, br2):
    n, d = x.shape
    assert d == D and n % NS == 0
    e = edge_index.shape[1]

    epw = -(-e // (NW * CB)) * CB
    ep = epw * NW
    npw = epw // CB
    pad = ep - e
    src_p = jnp.concatenate([edge_index[0], jnp.zeros((pad,), jnp.int32)])
    dst_p = jnp.concatenate([edge_index[1], jnp.full((pad,), n, jnp.int32)])

    agg, acc_rows = _make_agg(n, npw)
    cntk = _make_cnt(n, npw)
    comb = _make_combine(n, 1000)

    b1 = (bl1 + br1).reshape(1, D)
    b2 = (bl2 + br2).reshape(1, D)

    cnt = cntk(dst_p).reshape(NC, acc_rows, D)
    p1 = agg(x, src_p, dst_p).reshape(NC, acc_rows, D)
    y1 = comb(p1[0, :n], p1[1, :n], cnt[0, :n], cnt[1, :n], x,
              Wl1.T, Wr1.T, b1)
    p2 = agg(y1, src_p, dst_p).reshape(NC, acc_rows, D)
    y2 = comb(p2[0, :n], p2[1, :n], cnt[0, :n], cnt[1, :n], y1,
              Wl2.T, Wr2.T, b2)
    return (y2, edge_index, batch)
